# baseline (speedup 1.0000x reference)
import functools

import jax
import jax.numpy as jnp
from jax import lax
from jax.experimental import pallas as pl
from jax.experimental.pallas import tpu as pltpu
from jax.experimental.pallas import tpu_sc as plsc

D = 64
NC = 2
NS = 16
NW = NC * NS
LANES = 16
CHUNK = 128


@functools.partial(jax.jit, static_argnums=(3,))
def _gather_add(idx, embed_table, phoneme_table, n_rows):
    b_per_w = n_rows // NW
    n_chunks = b_per_w // CHUNK
    mesh = plsc.VectorSubcoreMesh(core_axis_name="c", subcore_axis_name="s")

    @functools.partial(
        pl.kernel,
        mesh=mesh,
        compiler_params=pltpu.CompilerParams(use_tc_tiling_on_sc=False),
        out_type=jax.ShapeDtypeStruct((n_rows, D), jnp.float32),
        scratch_types=[
            pltpu.VMEM((b_per_w,), jnp.int32),
            pltpu.VMEM((2, CHUNK, D), jnp.float32),
            pltpu.VMEM((2, CHUNK, D), jnp.float32),
            pltpu.SemaphoreType.DMA((2,)),
            pltpu.SemaphoreType.DMA((2,)),
        ],
    )
    def k(idx_hbm, embed_hbm, phon_hbm, out_hbm, idx_v, rows_e, rows_p,
          gsem, wsem):
        wid = lax.axis_index("s") * NC + lax.axis_index("c")
        base = wid * b_per_w

        pltpu.sync_copy(idx_hbm.at[pl.ds(base, b_per_w)], idx_v)

        def start_gathers(g, b):
            isl = idx_v.at[pl.ds(g * CHUNK, CHUNK)]
            pltpu.async_copy(embed_hbm.at[isl], rows_e.at[b], gsem.at[b])
            pltpu.async_copy(phon_hbm.at[isl], rows_p.at[b], gsem.at[b])

        def wait_gathers(b):
            isl = idx_v.at[pl.ds(0, CHUNK)]
            pltpu.make_async_copy(embed_hbm.at[isl], rows_e.at[b],
                                  gsem.at[b]).wait()
            pltpu.make_async_copy(phon_hbm.at[isl], rows_p.at[b],
                                  gsem.at[b]).wait()

        def wait_write(b):
            pltpu.make_async_copy(rows_e.at[b],
                                  out_hbm.at[pl.ds(base, CHUNK)],
                                  wsem.at[b]).wait()

        start_gathers(0, 0)

        def chunk_body(g, _):
            b = lax.rem(g, 2)
            nb = 1 - b

            @pl.when(g + 1 < n_chunks)
            def _():
                @pl.when(g >= 1)
                def _():
                    wait_write(nb)

                start_gathers(g + 1, nb)

            wait_gathers(b)

            def add_body(r, _):
                for c in range(D // LANES):
                    sl = pl.ds(c * LANES, LANES)
                    rows_e[b, r, sl] = rows_e[b, r, sl] + rows_p[b, r, sl]
                return ()

            lax.fori_loop(0, CHUNK, add_body, ())
            pltpu.async_copy(rows_e.at[b],
                             out_hbm.at[pl.ds(base + g * CHUNK, CHUNK)],
                             wsem.at[b])
            return ()

        lax.fori_loop(0, n_chunks, chunk_body, ())
        wait_write(0)
        wait_write(1)

    return k(idx, embed_table, phoneme_table)


def kernel(x, embed_table, phoneme_table):
    n_rows = x.shape[0] * x.shape[1]
    idx = x.reshape(n_rows).astype(jnp.int32)
    out = _gather_add(idx, embed_table, phoneme_table, n_rows)
    return out.reshape(x.shape[0], x.shape[1], D)


# speedup vs baseline: 1.1896x; 1.1896x over previous
import functools

import jax
import jax.numpy as jnp
from jax import lax
from jax.experimental import pallas as pl
from jax.experimental.pallas import tpu as pltpu
from jax.experimental.pallas import tpu_sc as plsc

D = 64
NC = 2
NS = 16
NW = NC * NS
CHUNK = 128


@functools.partial(jax.jit, static_argnums=(3,))
def _gather_add(idx, embed_table, phoneme_table, n_rows):
    b_per_w = n_rows // NW
    n_chunks = b_per_w // CHUNK
    mesh = plsc.VectorSubcoreMesh(core_axis_name="c", subcore_axis_name="s")

    @functools.partial(
        pl.kernel,
        mesh=mesh,
        compiler_params=pltpu.CompilerParams(use_tc_tiling_on_sc=False),
        out_type=jax.ShapeDtypeStruct((n_rows, D), jnp.float32),
        scratch_types=[
            pltpu.VMEM((b_per_w,), jnp.int32),
            pltpu.VMEM((2, CHUNK, D), jnp.float32),
            pltpu.SemaphoreType.DMA((2,)),
            pltpu.SemaphoreType.DMA((2,)),
            pltpu.SemaphoreType.DMA((2,)),
        ],
    )
    def k(idx_hbm, embed_hbm, phon_hbm, out_hbm, idx_v, rows,
          esem, psem, wsem):
        wid = lax.axis_index("s") * NC + lax.axis_index("c")
        base = wid * b_per_w

        pltpu.sync_copy(idx_hbm.at[pl.ds(base, b_per_w)], idx_v)

        def isl(g):
            return idx_v.at[pl.ds(g * CHUNK, CHUNK)]

        def wait(table, b, sem):
            pltpu.make_async_copy(table.at[isl(0)], rows.at[b],
                                  sem.at[b]).wait()

        def wait_write(b):
            pltpu.make_async_copy(rows.at[b],
                                  out_hbm.at[pl.ds(base, CHUNK)],
                                  wsem.at[b]).wait()

        pltpu.async_copy(embed_hbm.at[isl(0)], rows.at[0], esem.at[0])

        def chunk_body(g, _):
            b = lax.rem(g, 2)
            nb = 1 - b

            wait(embed_hbm, b, esem)
            pltpu.async_copy(phon_hbm.at[isl(g)], rows.at[b], psem.at[b],
                             add=True)

            @pl.when(g + 1 < n_chunks)
            def _():
                @pl.when(g >= 1)
                def _():
                    wait_write(nb)

                pltpu.async_copy(embed_hbm.at[isl(g + 1)], rows.at[nb],
                                 esem.at[nb])

            wait(phon_hbm, b, psem)
            pltpu.async_copy(rows.at[b],
                             out_hbm.at[pl.ds(base + g * CHUNK, CHUNK)],
                             wsem.at[b])
            return ()

        lax.fori_loop(0, n_chunks, chunk_body, ())
        wait_write(0)
        wait_write(1)

    return k(idx, embed_table, phoneme_table)


def kernel(x, embed_table, phoneme_table):
    n_rows = x.shape[0] * x.shape[1]
    idx = x.reshape(n_rows).astype(jnp.int32)
    out = _gather_add(idx, embed_table, phoneme_table, n_rows)
    return out.reshape(x.shape[0], x.shape[1], D)


# speedup vs baseline: 1.2273x; 1.0317x over previous
import functools

import jax
import jax.numpy as jnp
from jax import lax
from jax.experimental import pallas as pl
from jax.experimental.pallas import tpu as pltpu
from jax.experimental.pallas import tpu_sc as plsc

D = 64
NC = 2
NS = 16
NW = NC * NS
CHUNK = 256


@functools.partial(jax.jit, static_argnums=(3,))
def _gather_add(idx, embed_table, phoneme_table, n_rows):
    b_per_w = n_rows // NW
    n_chunks = b_per_w // CHUNK
    mesh = plsc.VectorSubcoreMesh(core_axis_name="c", subcore_axis_name="s")

    @functools.partial(
        pl.kernel,
        mesh=mesh,
        compiler_params=pltpu.CompilerParams(use_tc_tiling_on_sc=False),
        out_type=jax.ShapeDtypeStruct((n_rows, D), jnp.float32),
        scratch_types=[
            pltpu.VMEM((b_per_w,), jnp.int32),
            pltpu.VMEM((2, CHUNK, D), jnp.float32),
            pltpu.SemaphoreType.DMA((2,)),
            pltpu.SemaphoreType.DMA((2,)),
            pltpu.SemaphoreType.DMA((2,)),
        ],
    )
    def k(idx_hbm, embed_hbm, phon_hbm, out_hbm, idx_v, rows,
          esem, psem, wsem):
        wid = lax.axis_index("s") * NC + lax.axis_index("c")
        base = wid * b_per_w

        pltpu.sync_copy(idx_hbm.at[pl.ds(base, b_per_w)], idx_v)

        def isl(g):
            return idx_v.at[pl.ds(g * CHUNK, CHUNK)]

        def wait(table, b, sem):
            pltpu.make_async_copy(table.at[isl(0)], rows.at[b],
                                  sem.at[b]).wait()

        def wait_write(b):
            pltpu.make_async_copy(rows.at[b],
                                  out_hbm.at[pl.ds(base, CHUNK)],
                                  wsem.at[b]).wait()

        pltpu.async_copy(embed_hbm.at[isl(0)], rows.at[0], esem.at[0])

        def chunk_body(g, _):
            b = lax.rem(g, 2)
            nb = 1 - b

            wait(embed_hbm, b, esem)
            pltpu.async_copy(phon_hbm.at[isl(g)], rows.at[b], psem.at[b],
                             add=True)

            @pl.when(g + 1 < n_chunks)
            def _():
                @pl.when(g >= 1)
                def _():
                    wait_write(nb)

                pltpu.async_copy(embed_hbm.at[isl(g + 1)], rows.at[nb],
                                 esem.at[nb])

            wait(phon_hbm, b, psem)
            pltpu.async_copy(rows.at[b],
                             out_hbm.at[pl.ds(base + g * CHUNK, CHUNK)],
                             wsem.at[b])
            return ()

        lax.fori_loop(0, n_chunks, chunk_body, ())
        wait_write(0)
        wait_write(1)

    return k(idx, embed_table, phoneme_table)


def kernel(x, embed_table, phoneme_table):
    n_rows = x.shape[0] * x.shape[1]
    idx = x.reshape(n_rows).astype(jnp.int32)
    out = _gather_add(idx, embed_table, phoneme_table, n_rows)
    return out.reshape(x.shape[0], x.shape[1], D)


# speedup vs baseline: 1.2326x; 1.0043x over previous
import functools

import jax
import jax.numpy as jnp
from jax import lax
from jax.experimental import pallas as pl
from jax.experimental.pallas import tpu as pltpu
from jax.experimental.pallas import tpu_sc as plsc

D = 64
NC = 2
NS = 16
NW = NC * NS
CHUNK = 512


@functools.partial(jax.jit, static_argnums=(3,))
def _gather_add(idx, embed_table, phoneme_table, n_rows):
    b_per_w = n_rows // NW
    n_chunks = b_per_w // CHUNK
    mesh = plsc.VectorSubcoreMesh(core_axis_name="c", subcore_axis_name="s")

    @functools.partial(
        pl.kernel,
        mesh=mesh,
        compiler_params=pltpu.CompilerParams(use_tc_tiling_on_sc=False),
        out_type=jax.ShapeDtypeStruct((n_rows, D), jnp.float32),
        scratch_types=[
            pltpu.VMEM((b_per_w,), jnp.int32),
            pltpu.VMEM((2, CHUNK, D), jnp.float32),
            pltpu.SemaphoreType.DMA((2,)),
            pltpu.SemaphoreType.DMA((2,)),
            pltpu.SemaphoreType.DMA((2,)),
        ],
    )
    def k(idx_hbm, embed_hbm, phon_hbm, out_hbm, idx_v, rows,
          esem, psem, wsem):
        wid = lax.axis_index("s") * NC + lax.axis_index("c")
        base = wid * b_per_w

        pltpu.sync_copy(idx_hbm.at[pl.ds(base, b_per_w)], idx_v)

        def isl(g):
            return idx_v.at[pl.ds(g * CHUNK, CHUNK)]

        def wait(table, b, sem):
            pltpu.make_async_copy(table.at[isl(0)], rows.at[b],
                                  sem.at[b]).wait()

        def wait_write(b):
            pltpu.make_async_copy(rows.at[b],
                                  out_hbm.at[pl.ds(base, CHUNK)],
                                  wsem.at[b]).wait()

        pltpu.async_copy(embed_hbm.at[isl(0)], rows.at[0], esem.at[0])

        def chunk_body(g, _):
            b = lax.rem(g, 2)
            nb = 1 - b

            wait(embed_hbm, b, esem)
            pltpu.async_copy(phon_hbm.at[isl(g)], rows.at[b], psem.at[b],
                             add=True)

            @pl.when(g + 1 < n_chunks)
            def _():
                @pl.when(g >= 1)
                def _():
                    wait_write(nb)

                pltpu.async_copy(embed_hbm.at[isl(g + 1)], rows.at[nb],
                                 esem.at[nb])

            wait(phon_hbm, b, psem)
            pltpu.async_copy(rows.at[b],
                             out_hbm.at[pl.ds(base + g * CHUNK, CHUNK)],
                             wsem.at[b])
            return ()

        lax.fori_loop(0, n_chunks, chunk_body, ())
        wait_write(0)
        wait_write(1)

    return k(idx, embed_table, phoneme_table)


def kernel(x, embed_table, phoneme_table):
    n_rows = x.shape[0] * x.shape[1]
    idx = x.reshape(n_rows).astype(jnp.int32)
    out = _gather_add(idx, embed_table, phoneme_table, n_rows)
    return out.reshape(x.shape[0], x.shape[1], D)
